# baseline (speedup 1.0000x reference)
import functools

import jax
import jax.numpy as jnp
from jax import lax
from jax.experimental import pallas as pl
from jax.experimental.pallas import tpu as pltpu
from jax.experimental.pallas import tpu_sc as plsc

K = 5
KP = 8
THRESHOLD = 0.2
NC = 2
NS = 16
LANES = 128
DC = 8


def _score_topk_body(sel_ref, user_ref, kid_ref, w_ref):
    sel = sel_ref[...]
    u = user_ref[...]

    ssq_u = jnp.sum(u * u, axis=0, keepdims=True)
    un = u / jnp.maximum(jnp.sqrt(ssq_u), 1e-12)

    dot = jnp.sum(sel * un[None, None, :, :], axis=2)
    ssq = jnp.sum(sel * sel, axis=2)
    scores = dot / jnp.maximum(jnp.sqrt(ssq), 1e-12)

    s_len = scores.shape[1]
    iota = lax.broadcasted_iota(jnp.int32, scores.shape, 1)
    cur = scores
    kids, ws = [], []
    for _ in range(K):
        m = jnp.max(cur, axis=1, keepdims=True)
        idx = jnp.min(jnp.where(cur == m, iota, s_len),
                      axis=1, keepdims=True)
        oh = iota == idx
        ws.append(jnp.where(m < THRESHOLD, 0.0, m))
        kids.append(idx)
        cur = jnp.where(oh, -jnp.inf, cur)

    zi = jnp.zeros_like(kids[0])
    zf = jnp.zeros_like(ws[0])
    kid_ref[...] = jnp.concatenate(kids + [zi] * (KP - K), axis=1)
    w_ref[...] = jnp.concatenate(ws + [zf] * (KP - K), axis=1)


def _phase1(selT, userT, h0, nh):
    H, S, D, B = selT.shape
    Bl = 1024
    grid = (nh, B // Bl)
    return pl.pallas_call(
        _score_topk_body,
        grid=grid,
        in_specs=[
            pl.BlockSpec((1, S, D, Bl), lambda i, j: (h0 + i, 0, 0, j)),
            pl.BlockSpec((D, Bl), lambda i, j: (0, j)),
        ],
        out_specs=[
            pl.BlockSpec((1, KP, Bl), lambda i, j: (i, 0, j)),
            pl.BlockSpec((1, KP, Bl), lambda i, j: (i, 0, j)),
        ],
        out_shape=[
            jax.ShapeDtypeStruct((nh, KP, B), jnp.int32),
            jax.ShapeDtypeStruct((nh, KP, B), jnp.float32),
        ],
    )(selT, userT)


def _phase2(news6, kid4, w4, out_ref, h0):
    H, S, DHI, G, DLO, L = news6.shape
    NH = kid4.shape[0]
    n_tiles = NH * G
    n_workers = NC * NS
    n_jobs = n_tiles * DHI
    jobs_w = n_jobs // n_workers
    tiles_w = jobs_w // DHI + 1

    mesh = plsc.VectorSubcoreMesh(core_axis_name="c", subcore_axis_name="s")

    @functools.partial(
        pl.kernel,
        mesh=mesh,
        out_type=(),
        compiler_params=pltpu.CompilerParams(needs_layout_passes=False),
        scratch_types=[
            pltpu.VMEM((S, DLO, L), jnp.float32),
            pltpu.VMEM((S, DLO, L), jnp.float32),
            pltpu.VMEM((K, DLO, L), jnp.float32),
            pltpu.VMEM((K, DLO, L), jnp.float32),
            pltpu.VMEM((tiles_w, KP, L), jnp.int32),
            pltpu.VMEM((tiles_w, KP, L), jnp.float32),
            pltpu.SemaphoreType.DMA,
            pltpu.SemaphoreType.DMA,
            pltpu.SemaphoreType.DMA,
            pltpu.SemaphoreType.DMA,
        ],
    )
    def sc_gather(news_hbm, kid_hbm, w_hbm, out_hbm,
                  slab_a, slab_b, out_a, out_b, idx_all, w_all,
                  ssem_a, ssem_b, osem_a, osem_b):
        wid = lax.axis_index("s") * NC + lax.axis_index("c")
        jbase = wid * jobs_w
        t0 = jbase // DHI

        slabs = (slab_a, slab_b)
        outs = (out_a, out_b)
        ssems = (ssem_a, ssem_b)
        osems = (osem_a, osem_b)

        def job_hgd(jl):
            j = jbase + jl
            tile = j // DHI
            return h0 + tile // G, tile % G, j % DHI

        def start_slab(jl, par):
            h, g, dc = job_hgd(jl)
            pltpu.make_async_copy(
                news_hbm.at[h, :, dc, g], slabs[par], ssems[par]).start()

        def start_out(jl, par):
            h, g, dc = job_hgd(jl)
            pltpu.make_async_copy(
                outs[par], out_hbm.at[h, :, dc, g], osems[par]).start()

        def wait_slab(par):
            pltpu.make_async_copy(
                news_hbm.at[0, :, 0, 0], slabs[par], ssems[par]).wait()

        def wait_out(par):
            pltpu.make_async_copy(
                outs[par], out_hbm.at[0, :, 0, 0], osems[par]).wait()

        start_slab(0, 0)
        start_slab(1, 1)
        for i in range(tiles_w):
            t = t0 + i
            pltpu.sync_copy(kid_hbm.at[t // G, t % G], idx_all.at[i])
            pltpu.sync_copy(w_hbm.at[t // G, t % G], w_all.at[i])

        def pair_body(pair, _):
            for par in range(2):
                jl = pair * 2 + par
                til = (jbase + jl) // DHI - t0
                wait_slab(par)

                @pl.when(jl >= 2)
                def _wait_prev_out():
                    wait_out(par)

                slab_v = slabs[par]
                out_v = outs[par]
                for k in range(K):
                    for lg in range(L // 16):
                        sv = idx_all[til, k, pl.ds(lg * 16, 16)]
                        wv = w_all[til, k, pl.ds(lg * 16, 16)]
                        lane = jnp.arange(16, dtype=jnp.int32) + (lg * 16)
                        for dl in range(DLO):
                            dvec = jnp.full((16,), dl, dtype=jnp.int32)
                            g16 = plsc.load_gather(slab_v, [sv, dvec, lane])
                            out_v[k, dl, pl.ds(lg * 16, 16)] = g16 * wv
                start_out(jl, par)

                @pl.when(jl + 2 < jobs_w)
                def _prefetch_next():
                    start_slab(jl + 2, par)
            return _

        lax.fori_loop(0, jobs_w // 2, pair_body, 0)
        wait_out(0)
        wait_out(1)

    sc_gather(news6, kid4, w4, out_ref)


@jax.jit
def kernel(news_selection_embedding, news_embedding, user_repr):
    B, H, S, D = news_selection_embedding.shape
    selT = jnp.transpose(news_selection_embedding, (1, 2, 3, 0))
    userT = jnp.transpose(user_repr, (1, 2, 0))[0]

    G = B // LANES
    NSPLIT = 4
    NH = H // NSPLIT
    news6 = jnp.transpose(
        jnp.transpose(news_embedding, (1, 2, 3, 0))
        .reshape(H, S, D // 8, 8, G, LANES),
        (0, 1, 2, 4, 3, 5))

    def views(kid8, w8):
        kid4 = jnp.transpose(kid8.reshape(NH, KP, G, LANES), (0, 2, 1, 3))
        w4 = jnp.transpose(w8.reshape(NH, KP, G, LANES), (0, 2, 1, 3))
        return kid4, w4

    out_ref = jax.empty_ref(
        jax.ShapeDtypeStruct((H, K, D // 8, G, 8, LANES), jnp.float32))

    kid8s = []
    kid8, w8 = _phase1(selT, userT, 0, NH)
    for q in range(NSPLIT):
        kid8s.append(kid8)
        _phase2(news6, *views(kid8, w8), out_ref, q * NH)
        if q + 1 < NSPLIT:
            kid8, w8 = _phase1(selT, userT, (q + 1) * NH, NH)
    out6 = jax.freeze(out_ref)

    out = jnp.transpose(
        jnp.transpose(out6, (0, 1, 2, 4, 3, 5)).reshape(H, K, D, B),
        (3, 0, 1, 2))
    kid_full = jnp.concatenate(kid8s, axis=0)
    kid = jnp.transpose(kid_full[:, :K, :], (2, 0, 1))
    return (out, kid)


# speedup vs baseline: 1.0745x; 1.0745x over previous
import functools

import jax
import jax.numpy as jnp
from jax import lax
from jax.experimental import pallas as pl
from jax.experimental.pallas import tpu as pltpu
from jax.experimental.pallas import tpu_sc as plsc

K = 5
KP = 8
THRESHOLD = 0.2
NC = 2
NS = 16
LANES = 128
DC = 8


def _score_topk_body(sel_ref, user_ref, kid_ref, w_ref):
    sel = sel_ref[...]
    u = user_ref[...]

    ssq_u = jnp.sum(u * u, axis=0, keepdims=True)
    un = u / jnp.maximum(jnp.sqrt(ssq_u), 1e-12)

    dot = jnp.sum(sel * un[None, None, :, :], axis=2)
    ssq = jnp.sum(sel * sel, axis=2)
    scores = dot / jnp.maximum(jnp.sqrt(ssq), 1e-12)

    s_len = scores.shape[1]
    iota = lax.broadcasted_iota(jnp.int32, scores.shape, 1)
    cur = scores
    kids, ws = [], []
    for _ in range(K):
        m = jnp.max(cur, axis=1, keepdims=True)
        idx = jnp.min(jnp.where(cur == m, iota, s_len),
                      axis=1, keepdims=True)
        oh = iota == idx
        ws.append(jnp.where(m < THRESHOLD, 0.0, m))
        kids.append(idx)
        cur = jnp.where(oh, -jnp.inf, cur)

    zi = jnp.zeros_like(kids[0])
    zf = jnp.zeros_like(ws[0])
    kid_ref[...] = jnp.concatenate(kids + [zi] * (KP - K), axis=1)
    w_ref[...] = jnp.concatenate(ws + [zf] * (KP - K), axis=1)


def _phase1(selT, userT, h0, nh):
    H, S, D, B = selT.shape
    Bl = 1024
    grid = (nh, B // Bl)
    return pl.pallas_call(
        _score_topk_body,
        grid=grid,
        in_specs=[
            pl.BlockSpec((1, S, D, Bl), lambda i, j: (h0 + i, 0, 0, j)),
            pl.BlockSpec((D, Bl), lambda i, j: (0, j)),
        ],
        out_specs=[
            pl.BlockSpec((1, KP, Bl), lambda i, j: (i, 0, j)),
            pl.BlockSpec((1, KP, Bl), lambda i, j: (i, 0, j)),
        ],
        out_shape=[
            jax.ShapeDtypeStruct((nh, KP, B), jnp.int32),
            jax.ShapeDtypeStruct((nh, KP, B), jnp.float32),
        ],
    )(selT, userT)


def _phase2(news6, kid4, w4, out_ref, h0):
    H, S, DHI, G, DLO, L = news6.shape
    NH = kid4.shape[0]
    n_tiles = NH * G
    n_workers = NC * NS
    n_jobs = n_tiles * DHI
    jobs_w = n_jobs // n_workers
    tiles_w = jobs_w // DHI + 1

    mesh = plsc.VectorSubcoreMesh(core_axis_name="c", subcore_axis_name="s")

    @functools.partial(
        pl.kernel,
        mesh=mesh,
        out_type=(),
        compiler_params=pltpu.CompilerParams(needs_layout_passes=False),
        scratch_types=[
            pltpu.VMEM((S, DLO, L), jnp.float32),
            pltpu.VMEM((S, DLO, L), jnp.float32),
            pltpu.VMEM((K, DLO, L), jnp.float32),
            pltpu.VMEM((K, DLO, L), jnp.float32),
            pltpu.VMEM((tiles_w, KP, L), jnp.int32),
            pltpu.VMEM((tiles_w, KP, L), jnp.float32),
            pltpu.SemaphoreType.DMA,
            pltpu.SemaphoreType.DMA,
            pltpu.SemaphoreType.DMA,
            pltpu.SemaphoreType.DMA,
        ],
    )
    def sc_gather(news_hbm, kid_hbm, w_hbm, out_hbm,
                  slab_a, slab_b, out_a, out_b, idx_all, w_all,
                  ssem_a, ssem_b, osem_a, osem_b):
        wid = lax.axis_index("s") * NC + lax.axis_index("c")
        jbase = wid * jobs_w
        t0 = jbase // DHI

        slabs = (slab_a, slab_b)
        outs = (out_a, out_b)
        ssems = (ssem_a, ssem_b)
        osems = (osem_a, osem_b)

        def job_hgd(jl):
            j = jbase + jl
            tile = j // DHI
            return h0 + tile // G, tile % G, j % DHI

        def start_slab(jl, par):
            h, g, dc = job_hgd(jl)
            pltpu.make_async_copy(
                news_hbm.at[h, :, dc, g], slabs[par], ssems[par]).start()

        def start_out(jl, par):
            h, g, dc = job_hgd(jl)
            pltpu.make_async_copy(
                outs[par], out_hbm.at[h, :, dc, g], osems[par]).start()

        def wait_slab(par):
            pltpu.make_async_copy(
                news_hbm.at[0, :, 0, 0], slabs[par], ssems[par]).wait()

        def wait_out(par):
            pltpu.make_async_copy(
                outs[par], out_hbm.at[0, :, 0, 0], osems[par]).wait()

        start_slab(0, 0)
        start_slab(1, 1)
        for i in range(tiles_w):
            t = t0 + i
            pltpu.sync_copy(kid_hbm.at[t // G, t % G], idx_all.at[i])
            pltpu.sync_copy(w_hbm.at[t // G, t % G], w_all.at[i])

        def pair_body(pair, _):
            for par in range(2):
                jl = pair * 2 + par
                til = (jbase + jl) // DHI - t0
                wait_slab(par)

                @pl.when(jl >= 2)
                def _wait_prev_out():
                    wait_out(par)

                slab_v = slabs[par]
                out_v = outs[par]
                for k in range(K):
                    for lg in range(L // 16):
                        sv = idx_all[til, k, pl.ds(lg * 16, 16)]
                        wv = w_all[til, k, pl.ds(lg * 16, 16)]
                        lane = jnp.arange(16, dtype=jnp.int32) + (lg * 16)
                        for dl in range(DLO):
                            dvec = jnp.full((16,), dl, dtype=jnp.int32)
                            g16 = plsc.load_gather(slab_v, [sv, dvec, lane])
                            out_v[k, dl, pl.ds(lg * 16, 16)] = g16 * wv
                start_out(jl, par)

                @pl.when(jl + 2 < jobs_w)
                def _prefetch_next():
                    start_slab(jl + 2, par)
            return _

        lax.fori_loop(0, jobs_w // 2, pair_body, 0)
        wait_out(0)
        wait_out(1)

    sc_gather(news6, kid4, w4, out_ref)


@jax.jit
def kernel(news_selection_embedding, news_embedding, user_repr):
    B, H, S, D = news_selection_embedding.shape
    selT = jnp.transpose(news_selection_embedding, (1, 2, 3, 0))
    userT = jnp.transpose(user_repr, (1, 2, 0))[0]

    G = B // LANES
    NSPLIT = 2
    NH = H // NSPLIT
    news6 = jnp.transpose(
        jnp.transpose(news_embedding, (1, 2, 3, 0))
        .reshape(H, S, D // 8, 8, G, LANES),
        (0, 1, 2, 4, 3, 5))

    def views(kid8, w8):
        kid4 = jnp.transpose(kid8.reshape(NH, KP, G, LANES), (0, 2, 1, 3))
        w4 = jnp.transpose(w8.reshape(NH, KP, G, LANES), (0, 2, 1, 3))
        return kid4, w4

    out_ref = jax.empty_ref(
        jax.ShapeDtypeStruct((H, K, D // 8, G, 8, LANES), jnp.float32))

    kid8s = []
    kid8, w8 = _phase1(selT, userT, 0, NH)
    for q in range(NSPLIT):
        kid8s.append(kid8)
        _phase2(news6, *views(kid8, w8), out_ref, q * NH)
        if q + 1 < NSPLIT:
            kid8, w8 = _phase1(selT, userT, (q + 1) * NH, NH)
    out6 = jax.freeze(out_ref)

    out = jnp.transpose(
        jnp.transpose(out6, (0, 1, 2, 4, 3, 5)).reshape(H, K, D, B),
        (3, 0, 1, 2))
    kid_full = jnp.concatenate(kid8s, axis=0)
    kid = jnp.transpose(kid_full[:, :K, :], (2, 0, 1))
    return (out, kid)


# speedup vs baseline: 1.0821x; 1.0070x over previous
import functools
import math

import jax
import jax.numpy as jnp
from jax import lax
from jax.experimental import pallas as pl
from jax.experimental.pallas import tpu as pltpu
from jax.experimental.pallas import tpu_sc as plsc

K = 5
KP = 8
THRESHOLD = 0.2
NC = 2
NS = 16
LANES = 128
DC = 8


def _score_topk_body(sel_ref, user_ref, kid_ref, w_ref):
    sel = sel_ref[...]
    u = user_ref[...]

    ssq_u = jnp.sum(u * u, axis=0, keepdims=True)
    un = u / jnp.maximum(jnp.sqrt(ssq_u), 1e-12)

    dot = jnp.sum(sel * un[None, None, :, :], axis=2)
    ssq = jnp.sum(sel * sel, axis=2)
    scores = dot / jnp.maximum(jnp.sqrt(ssq), 1e-12)

    s_len = scores.shape[1]
    iota = lax.broadcasted_iota(jnp.int32, scores.shape, 1)
    cur = scores
    kids, ws = [], []
    for _ in range(K):
        m = jnp.max(cur, axis=1, keepdims=True)
        idx = jnp.min(jnp.where(cur == m, iota, s_len),
                      axis=1, keepdims=True)
        oh = iota == idx
        ws.append(jnp.where(m < THRESHOLD, 0.0, m))
        kids.append(idx)
        cur = jnp.where(oh, -jnp.inf, cur)

    zi = jnp.zeros_like(kids[0])
    zf = jnp.zeros_like(ws[0])
    kid_ref[...] = jnp.concatenate(kids + [zi] * (KP - K), axis=1)
    w_ref[...] = jnp.concatenate(ws + [zf] * (KP - K), axis=1)


def _phase1(selT, userT, h0, nh):
    H, S, D, B = selT.shape
    Bl = 1024
    grid = (nh, B // Bl)
    return pl.pallas_call(
        _score_topk_body,
        grid=grid,
        in_specs=[
            pl.BlockSpec((1, S, D, Bl), lambda i, j: (h0 + i, 0, 0, j)),
            pl.BlockSpec((D, Bl), lambda i, j: (0, j)),
        ],
        out_specs=[
            pl.BlockSpec((1, KP, Bl), lambda i, j: (i, 0, j)),
            pl.BlockSpec((1, KP, Bl), lambda i, j: (i, 0, j)),
        ],
        out_shape=[
            jax.ShapeDtypeStruct((nh, KP, B), jnp.int32),
            jax.ShapeDtypeStruct((nh, KP, B), jnp.float32),
        ],
    )(selT, userT)


def _phase2(news6, kid4, w4, out_ref, h0):
    H, S, DHI, G, DLO, L = news6.shape
    NH = kid4.shape[0]
    n_tiles = NH * G
    n_workers = NC * NS
    n_jobs = n_tiles * DHI
    jobs_w = n_jobs // n_workers
    assert jobs_w % 2 == 0
    g0 = math.gcd(jobs_w, DHI)
    tiles_w = ((DHI - g0) + jobs_w - 1) // DHI + 1

    mesh = plsc.VectorSubcoreMesh(core_axis_name="c", subcore_axis_name="s")

    @functools.partial(
        pl.kernel,
        mesh=mesh,
        out_type=(),
        compiler_params=pltpu.CompilerParams(needs_layout_passes=False),
        scratch_types=[
            pltpu.VMEM((S, DLO, L), jnp.float32),
            pltpu.VMEM((S, DLO, L), jnp.float32),
            pltpu.VMEM((K, DLO, L), jnp.float32),
            pltpu.VMEM((K, DLO, L), jnp.float32),
            pltpu.VMEM((tiles_w, KP, L), jnp.int32),
            pltpu.VMEM((tiles_w, KP, L), jnp.float32),
            pltpu.SemaphoreType.DMA,
            pltpu.SemaphoreType.DMA,
            pltpu.SemaphoreType.DMA,
            pltpu.SemaphoreType.DMA,
        ],
    )
    def sc_gather(news_hbm, kid_hbm, w_hbm, out_hbm,
                  slab_a, slab_b, out_a, out_b, idx_all, w_all,
                  ssem_a, ssem_b, osem_a, osem_b):
        wid = lax.axis_index("s") * NC + lax.axis_index("c")
        jbase = wid * jobs_w
        t0 = jbase // DHI

        slabs = (slab_a, slab_b)
        outs = (out_a, out_b)
        ssems = (ssem_a, ssem_b)
        osems = (osem_a, osem_b)

        def job_hgd(jl):
            j = jbase + jl
            tile = j // DHI
            return h0 + tile // G, tile % G, j % DHI

        def start_slab(jl, par):
            h, g, dc = job_hgd(jl)
            pltpu.make_async_copy(
                news_hbm.at[h, :, dc, g], slabs[par], ssems[par]).start()

        def start_out(jl, par):
            h, g, dc = job_hgd(jl)
            pltpu.make_async_copy(
                outs[par], out_hbm.at[h, :, dc, g], osems[par]).start()

        def wait_slab(par):
            pltpu.make_async_copy(
                news_hbm.at[0, :, 0, 0], slabs[par], ssems[par]).wait()

        def wait_out(par):
            pltpu.make_async_copy(
                outs[par], out_hbm.at[0, :, 0, 0], osems[par]).wait()

        start_slab(0, 0)
        start_slab(1, 1)
        for i in range(tiles_w):
            t = jnp.minimum(t0 + i, n_tiles - 1)
            pltpu.sync_copy(kid_hbm.at[t // G, t % G], idx_all.at[i])
            pltpu.sync_copy(w_hbm.at[t // G, t % G], w_all.at[i])

        def pair_body(pair, _):
            for par in range(2):
                jl = pair * 2 + par
                til = (jbase + jl) // DHI - t0
                wait_slab(par)

                @pl.when(jl >= 2)
                def _wait_prev_out():
                    wait_out(par)

                slab_v = slabs[par]
                out_v = outs[par]
                for k in range(K):
                    for lg in range(L // 16):
                        sv = idx_all[til, k, pl.ds(lg * 16, 16)]
                        wv = w_all[til, k, pl.ds(lg * 16, 16)]
                        lane = jnp.arange(16, dtype=jnp.int32) + (lg * 16)
                        for dl in range(DLO):
                            dvec = jnp.full((16,), dl, dtype=jnp.int32)
                            g16 = plsc.load_gather(slab_v, [sv, dvec, lane])
                            out_v[k, dl, pl.ds(lg * 16, 16)] = g16 * wv
                start_out(jl, par)

                @pl.when(jl + 2 < jobs_w)
                def _prefetch_next():
                    start_slab(jl + 2, par)
            return _

        lax.fori_loop(0, jobs_w // 2, pair_body, 0)
        wait_out(0)
        wait_out(1)

    sc_gather(news6, kid4, w4, out_ref)


@jax.jit
def kernel(news_selection_embedding, news_embedding, user_repr):
    B, H, S, D = news_selection_embedding.shape
    selT = jnp.transpose(news_selection_embedding, (1, 2, 3, 0))
    userT = jnp.transpose(user_repr, (1, 2, 0))[0]

    G = B // LANES
    H_SPLITS = (9, 11)
    news6 = jnp.transpose(
        jnp.transpose(news_embedding, (1, 2, 3, 0))
        .reshape(H, S, D // 8, 8, G, LANES),
        (0, 1, 2, 4, 3, 5))

    def views(kid8, w8):
        nh = kid8.shape[0]
        kid4 = jnp.transpose(kid8.reshape(nh, KP, G, LANES), (0, 2, 1, 3))
        w4 = jnp.transpose(w8.reshape(nh, KP, G, LANES), (0, 2, 1, 3))
        return kid4, w4

    out_ref = jax.empty_ref(
        jax.ShapeDtypeStruct((H, K, D // 8, G, 8, LANES), jnp.float32))

    kid8s = []
    h0 = 0
    kid8, w8 = _phase1(selT, userT, 0, H_SPLITS[0])
    for q, nh in enumerate(H_SPLITS):
        kid8s.append(kid8)
        _phase2(news6, *views(kid8, w8), out_ref, h0)
        h0 += nh
        if q + 1 < len(H_SPLITS):
            kid8, w8 = _phase1(selT, userT, h0, H_SPLITS[q + 1])
    out6 = jax.freeze(out_ref)

    out = jnp.transpose(
        jnp.transpose(out6, (0, 1, 2, 4, 3, 5)).reshape(H, K, D, B),
        (3, 0, 1, 2))
    kid_full = jnp.concatenate(kid8s, axis=0)
    kid = jnp.transpose(kid_full[:, :K, :], (2, 0, 1))
    return (out, kid)


# speedup vs baseline: 1.1172x; 1.0325x over previous
import functools
import math

import jax
import jax.numpy as jnp
from jax import lax
from jax.experimental import pallas as pl
from jax.experimental.pallas import tpu as pltpu
from jax.experimental.pallas import tpu_sc as plsc

K = 5
KP = 8
THRESHOLD = 0.2
NC = 2
NS = 16
LANES = 128
DC = 8


def _score_topk_body(sel_ref, user_ref, kid_ref, w_ref):
    sel = sel_ref[...]
    u = user_ref[...]

    ssq_u = jnp.sum(u * u, axis=0, keepdims=True)
    un = u / jnp.maximum(jnp.sqrt(ssq_u), 1e-12)

    dot = jnp.sum(sel * un[None, None, :, :], axis=2)
    ssq = jnp.sum(sel * sel, axis=2)
    scores = dot / jnp.maximum(jnp.sqrt(ssq), 1e-12)

    s_len = scores.shape[1]
    iota = lax.broadcasted_iota(jnp.int32, scores.shape, 1)
    cur = scores
    kids, ws = [], []
    for _ in range(K):
        m = jnp.max(cur, axis=1, keepdims=True)
        idx = jnp.min(jnp.where(cur == m, iota, s_len),
                      axis=1, keepdims=True)
        oh = iota == idx
        ws.append(jnp.where(m < THRESHOLD, 0.0, m))
        kids.append(idx)
        cur = jnp.where(oh, -jnp.inf, cur)

    zi = jnp.zeros_like(kids[0])
    zf = jnp.zeros_like(ws[0])
    kid_ref[...] = jnp.concatenate(kids + [zi] * (KP - K), axis=1)
    w_ref[...] = jnp.concatenate(ws + [zf] * (KP - K), axis=1)


def _phase1(selT, userT, h0, nh):
    H, S, D, B = selT.shape
    Bl = 1024
    grid = (nh, B // Bl)
    return pl.pallas_call(
        _score_topk_body,
        grid=grid,
        in_specs=[
            pl.BlockSpec((1, S, D, Bl), lambda i, j: (h0 + i, 0, 0, j)),
            pl.BlockSpec((D, Bl), lambda i, j: (0, j)),
        ],
        out_specs=[
            pl.BlockSpec((1, KP, Bl), lambda i, j: (i, 0, j)),
            pl.BlockSpec((1, KP, Bl), lambda i, j: (i, 0, j)),
        ],
        out_shape=[
            jax.ShapeDtypeStruct((nh, KP, B), jnp.int32),
            jax.ShapeDtypeStruct((nh, KP, B), jnp.float32),
        ],
    )(selT, userT)


def _phase2(news6, kid4, w4, out_ref, h0):
    H, S, DHI, G, DLO, L = news6.shape
    NH = kid4.shape[0]
    n_tiles = NH * G
    n_workers = NC * NS
    n_jobs = n_tiles * DHI
    jobs_w = n_jobs // n_workers
    assert jobs_w % 2 == 0
    g0 = math.gcd(jobs_w, DHI)
    tiles_w = ((DHI - g0) + jobs_w - 1) // DHI + 1

    mesh = plsc.VectorSubcoreMesh(core_axis_name="c", subcore_axis_name="s")

    @functools.partial(
        pl.kernel,
        mesh=mesh,
        out_type=(),
        compiler_params=pltpu.CompilerParams(needs_layout_passes=False),
        scratch_types=[
            pltpu.VMEM((S, DLO, L), jnp.float32),
            pltpu.VMEM((S, DLO, L), jnp.float32),
            pltpu.VMEM((K, DLO, L), jnp.float32),
            pltpu.VMEM((K, DLO, L), jnp.float32),
            pltpu.VMEM((tiles_w, KP, L), jnp.int32),
            pltpu.VMEM((tiles_w, KP, L), jnp.float32),
            pltpu.SemaphoreType.DMA,
            pltpu.SemaphoreType.DMA,
            pltpu.SemaphoreType.DMA,
            pltpu.SemaphoreType.DMA,
        ],
    )
    def sc_gather(news_hbm, kid_hbm, w_hbm, out_hbm,
                  slab_a, slab_b, out_a, out_b, idx_all, w_all,
                  ssem_a, ssem_b, osem_a, osem_b):
        wid = lax.axis_index("s") * NC + lax.axis_index("c")
        jbase = wid * jobs_w
        t0 = jbase // DHI

        slabs = (slab_a, slab_b)
        outs = (out_a, out_b)
        ssems = (ssem_a, ssem_b)
        osems = (osem_a, osem_b)

        def job_hgd(jl):
            j = jbase + jl
            tile = j // DHI
            return h0 + tile // G, tile % G, j % DHI

        def start_slab(jl, par):
            h, g, dc = job_hgd(jl)
            pltpu.make_async_copy(
                news_hbm.at[h, :, dc, g], slabs[par], ssems[par]).start()

        def start_out(jl, par):
            h, g, dc = job_hgd(jl)
            pltpu.make_async_copy(
                outs[par], out_hbm.at[h, :, dc, g], osems[par]).start()

        def wait_slab(par):
            pltpu.make_async_copy(
                news_hbm.at[0, :, 0, 0], slabs[par], ssems[par]).wait()

        def wait_out(par):
            pltpu.make_async_copy(
                outs[par], out_hbm.at[0, :, 0, 0], osems[par]).wait()

        start_slab(0, 0)
        start_slab(1, 1)
        for i in range(tiles_w):
            t = jnp.minimum(t0 + i, n_tiles - 1)
            pltpu.sync_copy(kid_hbm.at[t // G, t % G], idx_all.at[i])
            pltpu.sync_copy(w_hbm.at[t // G, t % G], w_all.at[i])

        def pair_body(pair, _):
            for par in range(2):
                jl = pair * 2 + par
                til = (jbase + jl) // DHI - t0
                wait_slab(par)

                @pl.when(jl >= 2)
                def _wait_prev_out():
                    wait_out(par)

                slab_v = slabs[par]
                out_v = outs[par]
                for k in range(K):
                    for lg in range(L // 16):
                        sv = idx_all[til, k, pl.ds(lg * 16, 16)]
                        wv = w_all[til, k, pl.ds(lg * 16, 16)]
                        lane = jnp.arange(16, dtype=jnp.int32) + (lg * 16)
                        for dl in range(DLO):
                            dvec = jnp.full((16,), dl, dtype=jnp.int32)
                            g16 = plsc.load_gather(slab_v, [sv, dvec, lane])
                            out_v[k, dl, pl.ds(lg * 16, 16)] = g16 * wv
                start_out(jl, par)

                @pl.when(jl + 2 < jobs_w)
                def _prefetch_next():
                    start_slab(jl + 2, par)
            return _

        lax.fori_loop(0, jobs_w // 2, pair_body, 0)
        wait_out(0)
        wait_out(1)

    sc_gather(news6, kid4, w4, out_ref)


@jax.jit
def kernel(news_selection_embedding, news_embedding, user_repr):
    B, H, S, D = news_selection_embedding.shape
    selT = jnp.transpose(news_selection_embedding, (1, 2, 3, 0))
    userT = jnp.transpose(user_repr, (1, 2, 0))[0]

    G = B // LANES
    H_SPLITS = (8, 12)
    news6 = jnp.transpose(
        jnp.transpose(news_embedding, (1, 2, 3, 0))
        .reshape(H, S, D // 8, 8, G, LANES),
        (0, 1, 2, 4, 3, 5))

    def views(kid8, w8):
        nh = kid8.shape[0]
        kid4 = jnp.transpose(kid8.reshape(nh, KP, G, LANES), (0, 2, 1, 3))
        w4 = jnp.transpose(w8.reshape(nh, KP, G, LANES), (0, 2, 1, 3))
        return kid4, w4

    out_ref = jax.empty_ref(
        jax.ShapeDtypeStruct((H, K, D // 8, G, 8, LANES), jnp.float32))

    kid8s = []
    h0 = 0
    kid8, w8 = _phase1(selT, userT, 0, H_SPLITS[0])
    for q, nh in enumerate(H_SPLITS):
        kid8s.append(kid8)
        _phase2(news6, *views(kid8, w8), out_ref, h0)
        h0 += nh
        if q + 1 < len(H_SPLITS):
            kid8, w8 = _phase1(selT, userT, h0, H_SPLITS[q + 1])
    out6 = jax.freeze(out_ref)

    out = jnp.transpose(
        jnp.transpose(out6, (0, 1, 2, 4, 3, 5)).reshape(H, K, D, B),
        (3, 0, 1, 2))
    kid_full = jnp.concatenate(kid8s, axis=0)
    kid = jnp.transpose(kid_full[:, :K, :], (2, 0, 1))
    return (out, kid)


# speedup vs baseline: 1.1235x; 1.0056x over previous
import functools
import math

import jax
import jax.numpy as jnp
from jax import lax
from jax.experimental import pallas as pl
from jax.experimental.pallas import tpu as pltpu
from jax.experimental.pallas import tpu_sc as plsc

K = 5
KP = 8
THRESHOLD = 0.2
NC = 2
NS = 16
LANES = 128
DC = 8


def _score_topk_body(sel_ref, user_ref, kid_ref, w_ref):
    sel = sel_ref[...]
    u = user_ref[...]

    ssq_u = jnp.sum(u * u, axis=0, keepdims=True)
    un = u / jnp.maximum(jnp.sqrt(ssq_u), 1e-12)

    dot = jnp.sum(sel * un[None, None, :, :], axis=2)
    ssq = jnp.sum(sel * sel, axis=2)
    scores = dot / jnp.maximum(jnp.sqrt(ssq), 1e-12)

    s_len = scores.shape[1]
    iota = lax.broadcasted_iota(jnp.int32, scores.shape, 1)
    cur = scores
    kids, ws = [], []
    for _ in range(K):
        m = jnp.max(cur, axis=1, keepdims=True)
        idx = jnp.min(jnp.where(cur == m, iota, s_len),
                      axis=1, keepdims=True)
        oh = iota == idx
        ws.append(jnp.where(m < THRESHOLD, 0.0, m))
        kids.append(idx)
        cur = jnp.where(oh, -jnp.inf, cur)

    zi = jnp.zeros_like(kids[0])
    zf = jnp.zeros_like(ws[0])
    kid_ref[...] = jnp.concatenate(kids + [zi] * (KP - K), axis=1)
    w_ref[...] = jnp.concatenate(ws + [zf] * (KP - K), axis=1)


def _phase1(selT, userT, h0, nh):
    H, S, D, B = selT.shape
    Bl = 1024
    grid = (nh, B // Bl)
    return pl.pallas_call(
        _score_topk_body,
        grid=grid,
        in_specs=[
            pl.BlockSpec((1, S, D, Bl), lambda i, j: (h0 + i, 0, 0, j)),
            pl.BlockSpec((D, Bl), lambda i, j: (0, j)),
        ],
        out_specs=[
            pl.BlockSpec((1, KP, Bl), lambda i, j: (i, 0, j)),
            pl.BlockSpec((1, KP, Bl), lambda i, j: (i, 0, j)),
        ],
        out_shape=[
            jax.ShapeDtypeStruct((nh, KP, B), jnp.int32),
            jax.ShapeDtypeStruct((nh, KP, B), jnp.float32),
        ],
    )(selT, userT)


def _phase2(news6, kid4, w4, out_ref, h0):
    H, S, DHI, G, DLO, L = news6.shape
    NH = kid4.shape[0]
    n_tiles = NH * G
    n_workers = NC * NS
    n_jobs = n_tiles * DHI
    jobs_w = n_jobs // n_workers
    assert jobs_w % 2 == 0
    g0 = math.gcd(jobs_w, DHI)
    tiles_w = ((DHI - g0) + jobs_w - 1) // DHI + 1

    mesh = plsc.VectorSubcoreMesh(core_axis_name="c", subcore_axis_name="s")

    @functools.partial(
        pl.kernel,
        mesh=mesh,
        out_type=(),
        compiler_params=pltpu.CompilerParams(needs_layout_passes=False),
        scratch_types=[
            pltpu.VMEM((S, DLO, L), jnp.float32),
            pltpu.VMEM((S, DLO, L), jnp.float32),
            pltpu.VMEM((K, DLO, L), jnp.float32),
            pltpu.VMEM((K, DLO, L), jnp.float32),
            pltpu.VMEM((tiles_w, KP, L), jnp.int32),
            pltpu.VMEM((tiles_w, KP, L), jnp.float32),
            pltpu.SemaphoreType.DMA,
            pltpu.SemaphoreType.DMA,
            pltpu.SemaphoreType.DMA,
            pltpu.SemaphoreType.DMA,
        ],
    )
    def sc_gather(news_hbm, kid_hbm, w_hbm, out_hbm,
                  slab_a, slab_b, out_a, out_b, idx_all, w_all,
                  ssem_a, ssem_b, osem_a, osem_b):
        wid = lax.axis_index("s") * NC + lax.axis_index("c")
        jbase = wid * jobs_w
        t0 = jbase // DHI

        slabs = (slab_a, slab_b)
        outs = (out_a, out_b)
        ssems = (ssem_a, ssem_b)
        osems = (osem_a, osem_b)

        def job_hgd(jl):
            j = jbase + jl
            tile = j // DHI
            return h0 + tile // G, tile % G, j % DHI

        def start_slab(jl, par):
            h, g, dc = job_hgd(jl)
            pltpu.make_async_copy(
                news_hbm.at[h, :, dc, g], slabs[par], ssems[par]).start()

        def start_out(jl, par):
            h, g, dc = job_hgd(jl)
            pltpu.make_async_copy(
                outs[par], out_hbm.at[h, :, dc, g], osems[par]).start()

        def wait_slab(par):
            pltpu.make_async_copy(
                news_hbm.at[0, :, 0, 0], slabs[par], ssems[par]).wait()

        def wait_out(par):
            pltpu.make_async_copy(
                outs[par], out_hbm.at[0, :, 0, 0], osems[par]).wait()

        start_slab(0, 0)
        start_slab(1, 1)
        for i in range(tiles_w):
            t = jnp.minimum(t0 + i, n_tiles - 1)
            pltpu.sync_copy(kid_hbm.at[t // G, t % G], idx_all.at[i])
            pltpu.sync_copy(w_hbm.at[t // G, t % G], w_all.at[i])

        def pair_body(pair, _):
            for par in range(2):
                jl = pair * 2 + par
                til = (jbase + jl) // DHI - t0
                wait_slab(par)

                @pl.when(jl >= 2)
                def _wait_prev_out():
                    wait_out(par)

                slab_v = slabs[par]
                out_v = outs[par]
                for k in range(K):
                    for lg in range(L // 16):
                        sv = idx_all[til, k, pl.ds(lg * 16, 16)]
                        wv = w_all[til, k, pl.ds(lg * 16, 16)]
                        lane = jnp.arange(16, dtype=jnp.int32) + (lg * 16)
                        for dl in range(DLO):
                            dvec = jnp.full((16,), dl, dtype=jnp.int32)
                            g16 = plsc.load_gather(slab_v, [sv, dvec, lane])
                            out_v[k, dl, pl.ds(lg * 16, 16)] = g16 * wv
                start_out(jl, par)

                @pl.when(jl + 2 < jobs_w)
                def _prefetch_next():
                    start_slab(jl + 2, par)
            return _

        lax.fori_loop(0, jobs_w // 2, pair_body, 0)
        wait_out(0)
        wait_out(1)

    sc_gather(news6, kid4, w4, out_ref)


@jax.jit
def kernel(news_selection_embedding, news_embedding, user_repr):
    B, H, S, D = news_selection_embedding.shape
    selT = jnp.transpose(news_selection_embedding, (1, 2, 3, 0))
    userT = jnp.transpose(user_repr, (1, 2, 0))[0]

    G = B // LANES
    H_SPLITS = (7, 13)
    news6 = jnp.transpose(
        jnp.transpose(news_embedding, (1, 2, 3, 0))
        .reshape(H, S, D // 8, 8, G, LANES),
        (0, 1, 2, 4, 3, 5))

    def views(kid8, w8):
        nh = kid8.shape[0]
        kid4 = jnp.transpose(kid8.reshape(nh, KP, G, LANES), (0, 2, 1, 3))
        w4 = jnp.transpose(w8.reshape(nh, KP, G, LANES), (0, 2, 1, 3))
        return kid4, w4

    out_ref = jax.empty_ref(
        jax.ShapeDtypeStruct((H, K, D // 8, G, 8, LANES), jnp.float32))

    kid8s = []
    h0 = 0
    kid8, w8 = _phase1(selT, userT, 0, H_SPLITS[0])
    for q, nh in enumerate(H_SPLITS):
        kid8s.append(kid8)
        _phase2(news6, *views(kid8, w8), out_ref, h0)
        h0 += nh
        if q + 1 < len(H_SPLITS):
            kid8, w8 = _phase1(selT, userT, h0, H_SPLITS[q + 1])
    out6 = jax.freeze(out_ref)

    out = jnp.transpose(
        jnp.transpose(out6, (0, 1, 2, 4, 3, 5)).reshape(H, K, D, B),
        (3, 0, 1, 2))
    kid_full = jnp.concatenate(kid8s, axis=0)
    kid = jnp.transpose(kid_full[:, :K, :], (2, 0, 1))
    return (out, kid)
